# baseline (speedup 1.0000x reference)
from functools import partial

import jax
import jax.numpy as jnp
from jax.experimental import pallas as pl
from jax.experimental.pallas import tpu as pltpu


def _conv_stats_kernel(x_ref, w_ref, b_ref, o_ref, sum_ref, ssq_ref, *, H, W):
    xb = x_ref[0]
    lane = jax.lax.broadcasted_iota(jnp.int32, xb.shape, 1)
    col = lane % W

    taps = {}
    for dh in (-1, 0, 1):
        for dw in (-1, 0, 1):
            k = dh * W + dw
            t = xb if k == 0 else jnp.roll(xb, -k, axis=1)
            masks = []
            if dh == 1:
                masks.append(lane < (H - 1) * W)
            elif dh == -1:
                masks.append(lane >= W)
            if dw == 1:
                masks.append(col < (W - 1))
            elif dw == -1:
                masks.append(col >= 1)
            if masks:
                m = masks[0]
                for mm in masks[1:]:
                    m = jnp.logical_and(m, mm)
                t = jnp.where(m, t, 0.0)
            taps[(dh, dw)] = t

    b = b_ref[...]
    ssum = jnp.zeros_like(b)
    ssq = jnp.zeros_like(b)
    p = 0
    for rh in (0, 1):
        ch = 1 - rh
        for rw in (0, 1):
            cw = 1 - rw
            acc = None
            for mh in (0, 1):
                for mw in (0, 1):
                    d = jnp.dot(w_ref[p, mh * 2 + mw],
                                taps[(ch - mh, cw - mw)],
                                preferred_element_type=jnp.float32)
                    acc = d if acc is None else acc + d
            y = acc + b
            o_ref[0, p] = y
            ssum = ssum + jnp.sum(y, axis=1, keepdims=True)
            ssq = ssq + jnp.sum(y * y, axis=1, keepdims=True)
            p += 1
    sum_ref[0] = ssum
    ssq_ref[0] = ssq


def _bn_relu_kernel(c_ref, sc_ref, sh_ref, o_ref):
    o_ref[0] = jnp.maximum(c_ref[0] * sc_ref[...] + sh_ref[...], 0.0)


def _upsample(x, w_t, bias, gamma, beta, *, eps=1e-5):
    N, Cin, H, W = map(int, x.shape)
    _, Cout, K, _ = map(int, w_t.shape)
    assert K == 4
    HW = H * W
    P = 4

    xf = x.reshape(N, Cin, HW)

    wp = []
    for rh in (0, 1):
        for rw in (0, 1):
            w_sub = w_t[:, :, rh::2, rw::2]
            wp.append(jnp.stack(
                [w_sub[:, :, mh, mw].T for mh in (0, 1) for mw in (0, 1)], 0))
    wms = jnp.stack(wp, 0).astype(jnp.float32)
    b2 = bias.reshape(Cout, 1).astype(jnp.float32)

    conv, sums, ssq = pl.pallas_call(
        partial(_conv_stats_kernel, H=H, W=W),
        out_shape=(
            jax.ShapeDtypeStruct((N, P, Cout, HW), jnp.float32),
            jax.ShapeDtypeStruct((N, Cout, 1), jnp.float32),
            jax.ShapeDtypeStruct((N, Cout, 1), jnp.float32),
        ),
        grid=(N,),
        in_specs=[
            pl.BlockSpec((1, Cin, HW), lambda n: (n, 0, 0)),
            pl.BlockSpec((P, 4, Cout, Cin), lambda n: (0, 0, 0, 0)),
            pl.BlockSpec((Cout, 1), lambda n: (0, 0)),
        ],
        out_specs=(
            pl.BlockSpec((1, P, Cout, HW), lambda n: (n, 0, 0, 0)),
            pl.BlockSpec((1, Cout, 1), lambda n: (n, 0, 0)),
            pl.BlockSpec((1, Cout, 1), lambda n: (n, 0, 0)),
        ),
        compiler_params=pltpu.CompilerParams(
            dimension_semantics=("parallel",)),
    )(xf, wms, b2)

    Mtot = float(N * P * HW)
    mean = jnp.sum(sums, axis=0) / Mtot
    var = jnp.maximum(jnp.sum(ssq, axis=0) / Mtot - mean * mean, 0.0)
    inv = jax.lax.rsqrt(var + eps)
    scale = gamma.reshape(Cout, 1).astype(jnp.float32) * inv
    shift = beta.reshape(Cout, 1).astype(jnp.float32) - mean * scale

    z = pl.pallas_call(
        _bn_relu_kernel,
        out_shape=jax.ShapeDtypeStruct((N, P, Cout, HW), jnp.float32),
        grid=(N,),
        in_specs=[
            pl.BlockSpec((1, P, Cout, HW), lambda n: (n, 0, 0, 0)),
            pl.BlockSpec((Cout, 1), lambda n: (0, 0)),
            pl.BlockSpec((Cout, 1), lambda n: (0, 0)),
        ],
        out_specs=pl.BlockSpec((1, P, Cout, HW), lambda n: (n, 0, 0, 0)),
        compiler_params=pltpu.CompilerParams(
            dimension_semantics=("parallel",)),
    )(conv, scale, shift)

    z5 = z.reshape(N, 2, 2, Cout, H, W)
    jv = jax.lax.broadcasted_iota(jnp.int32, (2, W, 2 * W), 2)
    S = (jv == 2 * jax.lax.broadcasted_iota(jnp.int32, (2, W, 2 * W), 1)
         + jax.lax.broadcasted_iota(jnp.int32, (2, W, 2 * W), 0)
         ).astype(jnp.float32)
    iu = jax.lax.broadcasted_iota(jnp.int32, (2, H, 2 * H), 2)
    R = (iu == 2 * jax.lax.broadcasted_iota(jnp.int32, (2, H, 2 * H), 1)
         + jax.lax.broadcasted_iota(jnp.int32, (2, H, 2 * H), 0)
         ).astype(jnp.float32)
    out = 0.0
    for a in (0, 1):
        m = (jnp.einsum('nchw,wv->nchv', z5[:, 1 - a, 1], S[0])
             + jnp.einsum('nchw,wv->nchv', z5[:, 1 - a, 0], S[1]))
        out = out + jnp.einsum('nchv,hu->ncuv', m, R[a])
    return out


def kernel(x, w_t, bias, gamma, beta):
    return _upsample(x, w_t, bias, gamma, beta)


# speedup vs baseline: 2.6080x; 2.6080x over previous
from functools import partial

import jax
import jax.numpy as jnp
from jax.experimental import pallas as pl
from jax.experimental.pallas import tpu as pltpu


def _conv_stats_kernel(x_ref, w_ref, b_ref, o_ref, sum_ref, ssq_ref, *, H, W):
    xb = x_ref[0]
    lane = jax.lax.broadcasted_iota(jnp.int32, xb.shape, 1)
    col = lane % W

    taps = {}
    for dh in (-1, 0, 1):
        for dw in (-1, 0, 1):
            k = dh * W + dw
            t = xb if k == 0 else jnp.roll(xb, -k, axis=1)
            masks = []
            if dh == 1:
                masks.append(lane < (H - 1) * W)
            elif dh == -1:
                masks.append(lane >= W)
            if dw == 1:
                masks.append(col < (W - 1))
            elif dw == -1:
                masks.append(col >= 1)
            if masks:
                m = masks[0]
                for mm in masks[1:]:
                    m = jnp.logical_and(m, mm)
                t = jnp.where(m, t, 0.0)
            taps[(dh, dw)] = t

    b = b_ref[...]
    ssum = jnp.zeros_like(b)
    ssq = jnp.zeros_like(b)
    p = 0
    for rh in (0, 1):
        ch = 1 - rh
        for rw in (0, 1):
            cw = 1 - rw
            acc = None
            for mh in (0, 1):
                for mw in (0, 1):
                    d = jnp.dot(w_ref[p, mh * 2 + mw],
                                taps[(ch - mh, cw - mw)],
                                preferred_element_type=jnp.float32)
                    acc = d if acc is None else acc + d
            y = acc + b
            o_ref[0, p] = y
            ssum = ssum + jnp.sum(y, axis=1, keepdims=True)
            ssq = ssq + jnp.sum(y * y, axis=1, keepdims=True)
            p += 1
    sum_ref[0] = ssum
    ssq_ref[0] = ssq


def _bn_relu_interleave_kernel(c_ref, sc_ref, sh_ref, o_ref, *, H, W):
    y = jnp.maximum(c_ref[0] * sc_ref[...] + sh_ref[...], 0.0)
    Cout = y.shape[1]
    src = jnp.concatenate(
        [y[p].reshape(Cout, H, W) for p in range(4)], axis=-1)
    g = jax.lax.broadcasted_iota(jnp.int32, (Cout, H, 4 * W), 2)
    half = g // (2 * W)
    gg = g % (2 * W)
    p = 2 * (1 - half) + (1 - gg % 2)
    idx = p * W + gg // 2
    o_ref[0] = jnp.take_along_axis(src, idx, axis=-1)


def _upsample(x, w_t, bias, gamma, beta, *, eps=1e-5):
    N, Cin, H, W = map(int, x.shape)
    _, Cout, K, _ = map(int, w_t.shape)
    assert K == 4
    HW = H * W
    P = 4

    xf = x.reshape(N, Cin, HW)

    wp = []
    for rh in (0, 1):
        for rw in (0, 1):
            w_sub = w_t[:, :, rh::2, rw::2]
            wp.append(jnp.stack(
                [w_sub[:, :, mh, mw].T for mh in (0, 1) for mw in (0, 1)], 0))
    wms = jnp.stack(wp, 0).astype(jnp.float32)
    b2 = bias.reshape(Cout, 1).astype(jnp.float32)

    conv, sums, ssq = pl.pallas_call(
        partial(_conv_stats_kernel, H=H, W=W),
        out_shape=(
            jax.ShapeDtypeStruct((N, P, Cout, HW), jnp.float32),
            jax.ShapeDtypeStruct((N, Cout, 1), jnp.float32),
            jax.ShapeDtypeStruct((N, Cout, 1), jnp.float32),
        ),
        grid=(N,),
        in_specs=[
            pl.BlockSpec((1, Cin, HW), lambda n: (n, 0, 0)),
            pl.BlockSpec((P, 4, Cout, Cin), lambda n: (0, 0, 0, 0)),
            pl.BlockSpec((Cout, 1), lambda n: (0, 0)),
        ],
        out_specs=(
            pl.BlockSpec((1, P, Cout, HW), lambda n: (n, 0, 0, 0)),
            pl.BlockSpec((1, Cout, 1), lambda n: (n, 0, 0)),
            pl.BlockSpec((1, Cout, 1), lambda n: (n, 0, 0)),
        ),
        compiler_params=pltpu.CompilerParams(
            dimension_semantics=("parallel",)),
    )(xf, wms, b2)

    Mtot = float(N * P * HW)
    mean = jnp.sum(sums, axis=0) / Mtot
    var = jnp.maximum(jnp.sum(ssq, axis=0) / Mtot - mean * mean, 0.0)
    inv = jax.lax.rsqrt(var + eps)
    scale = gamma.reshape(Cout, 1).astype(jnp.float32) * inv
    shift = beta.reshape(Cout, 1).astype(jnp.float32) - mean * scale

    out = pl.pallas_call(
        partial(_bn_relu_interleave_kernel, H=H, W=W),
        out_shape=jax.ShapeDtypeStruct((N, Cout, H, 4 * W), jnp.float32),
        grid=(N,),
        in_specs=[
            pl.BlockSpec((1, P, Cout, HW), lambda n: (n, 0, 0, 0)),
            pl.BlockSpec((Cout, 1), lambda n: (0, 0)),
            pl.BlockSpec((Cout, 1), lambda n: (0, 0)),
        ],
        out_specs=pl.BlockSpec((1, Cout, H, 4 * W), lambda n: (n, 0, 0, 0)),
        compiler_params=pltpu.CompilerParams(
            dimension_semantics=("parallel",)),
    )(conv, scale, shift)

    return out.reshape(N, Cout, 2 * H, 2 * W)


def kernel(x, w_t, bias, gamma, beta):
    return _upsample(x, w_t, bias, gamma, beta)


# speedup vs baseline: 6.0702x; 2.3275x over previous
from functools import partial

import jax
import jax.numpy as jnp
from jax.experimental import pallas as pl
from jax.experimental.pallas import tpu as pltpu


def _conv_stats_kernel(x_ref, w_ref, b_ref, o_ref, sum_ref, ssq_ref, *, H, W):
    xb = x_ref[0]
    lane = jax.lax.broadcasted_iota(jnp.int32, xb.shape, 1)
    col = lane % W

    taps = {}
    for dh in (-1, 0, 1):
        for dw in (-1, 0, 1):
            k = dh * W + dw
            t = xb if k == 0 else jnp.roll(xb, -k, axis=1)
            masks = []
            if dh == 1:
                masks.append(lane < (H - 1) * W)
            elif dh == -1:
                masks.append(lane >= W)
            if dw == 1:
                masks.append(col < (W - 1))
            elif dw == -1:
                masks.append(col >= 1)
            if masks:
                m = masks[0]
                for mm in masks[1:]:
                    m = jnp.logical_and(m, mm)
                t = jnp.where(m, t, 0.0)
            taps[(dh, dw)] = t

    b = b_ref[...]
    ssum = jnp.zeros_like(b)
    ssq = jnp.zeros_like(b)
    p = 0
    for rh in (0, 1):
        ch = 1 - rh
        for rw in (0, 1):
            cw = 1 - rw
            acc = None
            for mh in (0, 1):
                for mw in (0, 1):
                    d = jnp.dot(w_ref[p, mh * 2 + mw],
                                taps[(ch - mh, cw - mw)],
                                preferred_element_type=jnp.float32)
                    acc = d if acc is None else acc + d
            y = acc + b
            o_ref[0, p] = y
            ssum = ssum + jnp.sum(y, axis=1, keepdims=True)
            ssq = ssq + jnp.sum(y * y, axis=1, keepdims=True)
            p += 1
    sum_ref[0] = ssum
    ssq_ref[0] = ssq


def _bn_relu_interleave_kernel(c_ref, sc_ref, sh_ref, o_ref, *, H, W):
    y = jnp.maximum(c_ref[0] * sc_ref[...] + sh_ref[...], 0.0)
    Cout = y.shape[1]
    src = jnp.concatenate(
        [y[p].reshape(Cout, H, W) for p in range(4)], axis=-1)
    g = jax.lax.broadcasted_iota(jnp.int32, (Cout, H, 4 * W), 2)
    half = g // (2 * W)
    gg = g % (2 * W)
    p = 2 * (1 - half) + (1 - gg % 2)
    idx = p * W + gg // 2
    o_ref[0] = jnp.take_along_axis(src, idx, axis=-1)


def _upsample(x, w_t, bias, gamma, beta, *, eps=1e-5):
    N, Cin, H, W = map(int, x.shape)
    _, Cout, K, _ = map(int, w_t.shape)
    assert K == 4
    HW = H * W
    P = 4

    xf = x.reshape(N, Cin, HW)

    wms = (w_t.transpose(2, 3, 1, 0)
           .reshape(2, 2, 2, 2, Cout, Cin)
           .transpose(1, 3, 0, 2, 4, 5)
           .reshape(P, 4, Cout, Cin).astype(jnp.float32))
    b2 = bias.reshape(Cout, 1).astype(jnp.float32)

    conv, sums, ssq = pl.pallas_call(
        partial(_conv_stats_kernel, H=H, W=W),
        out_shape=(
            jax.ShapeDtypeStruct((N, P, Cout, HW), jnp.float32),
            jax.ShapeDtypeStruct((N, Cout, 1), jnp.float32),
            jax.ShapeDtypeStruct((N, Cout, 1), jnp.float32),
        ),
        grid=(N,),
        in_specs=[
            pl.BlockSpec((1, Cin, HW), lambda n: (n, 0, 0)),
            pl.BlockSpec((P, 4, Cout, Cin), lambda n: (0, 0, 0, 0)),
            pl.BlockSpec((Cout, 1), lambda n: (0, 0)),
        ],
        out_specs=(
            pl.BlockSpec((1, P, Cout, HW), lambda n: (n, 0, 0, 0)),
            pl.BlockSpec((1, Cout, 1), lambda n: (n, 0, 0)),
            pl.BlockSpec((1, Cout, 1), lambda n: (n, 0, 0)),
        ),
        compiler_params=pltpu.CompilerParams(
            dimension_semantics=("parallel",)),
    )(xf, wms, b2)

    Mtot = float(N * P * HW)
    mean = jnp.sum(sums, axis=0) / Mtot
    var = jnp.maximum(jnp.sum(ssq, axis=0) / Mtot - mean * mean, 0.0)
    inv = jax.lax.rsqrt(var + eps)
    scale = gamma.reshape(Cout, 1).astype(jnp.float32) * inv
    shift = beta.reshape(Cout, 1).astype(jnp.float32) - mean * scale

    out = pl.pallas_call(
        partial(_bn_relu_interleave_kernel, H=H, W=W),
        out_shape=jax.ShapeDtypeStruct((N, Cout, H, 4 * W), jnp.float32),
        grid=(N,),
        in_specs=[
            pl.BlockSpec((1, P, Cout, HW), lambda n: (n, 0, 0, 0)),
            pl.BlockSpec((Cout, 1), lambda n: (0, 0)),
            pl.BlockSpec((Cout, 1), lambda n: (0, 0)),
        ],
        out_specs=pl.BlockSpec((1, Cout, H, 4 * W), lambda n: (n, 0, 0, 0)),
        compiler_params=pltpu.CompilerParams(
            dimension_semantics=("parallel",)),
    )(conv, scale, shift)

    return out.reshape(N, Cout, 2 * H, 2 * W)


def kernel(x, w_t, bias, gamma, beta):
    return _upsample(x, w_t, bias, gamma, beta)
